# speedup vs baseline: 7.6079x; 7.6079x over previous
import functools

import jax
import jax.numpy as jnp
import numpy as np
from jax import lax
from jax.experimental import pallas as pl
from jax.experimental.pallas import tpu as pltpu
from jax.experimental.pallas import tpu_sc as plsc

N = 10000
E = 320000
D = 128
G = 64
NC, NS = 2, 16
NP = 10240
R = 1024
EB = 128
K = 79
EP = NC * NS * K * EB
ROWS_PER_TILE = NP // NS


def _agg_body(h_hbm, src_hbm, dst_hbm, zeros_hbm, out_hbm,
              acc, src_v, dst_v, rows_v, sem):
    cid = lax.axis_index("c")
    sid = lax.axis_index("s")
    pltpu.sync_copy(zeros_hbm, acc.at[pl.ds(sid * ROWS_PER_TILE, ROWS_PER_TILE)])
    pltpu.sync_copy(src_hbm.at[cid, sid], src_v)
    pltpu.sync_copy(dst_hbm.at[cid, sid], dst_v)
    plsc.subcore_barrier()

    def step(j, carry):
        pltpu.async_copy(h_hbm.at[src_v.at[j]], rows_v, sem).wait()
        pltpu.sync_copy(rows_v, acc.at[dst_v.at[j]], add=True)
        return carry

    lax.fori_loop(0, K, step, 0)
    plsc.subcore_barrier()
    pltpu.sync_copy(acc.at[pl.ds(sid * ROWS_PER_TILE, ROWS_PER_TILE)],
                    out_hbm.at[cid, pl.ds(sid * ROWS_PER_TILE, ROWS_PER_TILE)])


_agg = functools.partial(
    pl.kernel,
    out_type=jax.ShapeDtypeStruct((NC, NP, D), jnp.float32),
    mesh=plsc.VectorSubcoreMesh(core_axis_name="c", subcore_axis_name="s"),
    scratch_types=[
        pltpu.VMEM_SHARED((NP, D), jnp.float32),
        pltpu.VMEM((K, EB), jnp.int32),
        pltpu.VMEM((K, EB), jnp.int32),
        pltpu.VMEM((EB, D), jnp.float32),
        pltpu.SemaphoreType.DMA,
    ],
)(_agg_body)


def _mlp_body(h_ref, p_ref, w1_ref, b1_ref, w2_ref, b2_ref, o_ref):
    z = h_ref[...] + p_ref[0] + p_ref[1]
    z = jnp.maximum(
        jnp.dot(z, w1_ref[...], preferred_element_type=jnp.float32) + b1_ref[...],
        0.0)
    z = jnp.dot(z, w2_ref[...], preferred_element_type=jnp.float32) + b2_ref[...]
    o_ref[...] = jnp.maximum(z, 0.0)


_mlp = pl.pallas_call(
    _mlp_body,
    grid=(NP // R,),
    in_specs=[
        pl.BlockSpec((R, D), lambda i: (i, 0)),
        pl.BlockSpec((NC, R, D), lambda i: (0, i, 0)),
        pl.BlockSpec((D, D), lambda i: (0, 0)),
        pl.BlockSpec((1, D), lambda i: (0, 0)),
        pl.BlockSpec((D, D), lambda i: (0, 0)),
        pl.BlockSpec((1, D), lambda i: (0, 0)),
    ],
    out_specs=pl.BlockSpec((R, D), lambda i: (i, 0)),
    out_shape=jax.ShapeDtypeStruct((NP, D), jnp.float32),
)


def _mlp_pool_body(h_ref, p_ref, batch_ref, w1_ref, b1_ref, w2_ref, b2_ref,
                   wf1_ref, bf1_ref, wf2_ref, bf2_ref, o_ref, pooled):
    i = pl.program_id(0)
    z = h_ref[...] + p_ref[0] + p_ref[1]
    z = jnp.maximum(
        jnp.dot(z, w1_ref[...], preferred_element_type=jnp.float32) + b1_ref[...],
        0.0)
    z = jnp.dot(z, w2_ref[...], preferred_element_type=jnp.float32) + b2_ref[...]
    h3 = jnp.maximum(z, 0.0)
    seg = batch_ref[0, 0, :]
    m = (lax.broadcasted_iota(jnp.int32, (G, R), 0) == seg[None, :]
         ).astype(jnp.float32)
    part = jnp.dot(m, h3, preferred_element_type=jnp.float32)

    @pl.when(i == 0)
    def _():
        pooled[...] = part

    @pl.when(i > 0)
    def _():
        pooled[...] += part

    @pl.when(i == pl.num_programs(0) - 1)
    def _():
        q = jnp.maximum(
            jnp.dot(pooled[...], wf1_ref[...],
                    preferred_element_type=jnp.float32) + bf1_ref[...],
            0.0)
        o_ref[...] = (jnp.dot(q, wf2_ref[...],
                              preferred_element_type=jnp.float32)
                      + bf2_ref[...])


_mlp_pool = pl.pallas_call(
    _mlp_pool_body,
    grid=(NP // R,),
    in_specs=[
        pl.BlockSpec((R, D), lambda i: (i, 0)),
        pl.BlockSpec((NC, R, D), lambda i: (0, i, 0)),
        pl.BlockSpec((1, 1, R), lambda i: (i, 0, 0)),
        pl.BlockSpec((D, D), lambda i: (0, 0)),
        pl.BlockSpec((1, D), lambda i: (0, 0)),
        pl.BlockSpec((D, D), lambda i: (0, 0)),
        pl.BlockSpec((1, D), lambda i: (0, 0)),
        pl.BlockSpec((D, D), lambda i: (0, 0)),
        pl.BlockSpec((1, D), lambda i: (0, 0)),
        pl.BlockSpec((D, D), lambda i: (0, 0)),
        pl.BlockSpec((1, D), lambda i: (0, 0)),
    ],
    out_specs=pl.BlockSpec((G, D), lambda i: (0, 0)),
    out_shape=jax.ShapeDtypeStruct((G, D), jnp.float32),
    scratch_shapes=[pltpu.VMEM((G, D), jnp.float32)],
)


def kernel(x, edge_index, edge_attr, batch,
           W1_0, b1_0, W2_0, b2_0, W1_1, b1_1, W2_1, b2_1,
           W1_2, b1_2, W2_2, b2_2, Wf1, bf1, Wf2, bf2):
    del edge_attr
    pad = EP - E
    pad_src = (np.arange(pad, dtype=np.int32) * 97) % N
    pad_dst = N + (np.arange(pad, dtype=np.int32) % (NP - N))
    src = jnp.concatenate([edge_index[0].astype(jnp.int32), jnp.asarray(pad_src)])
    dst = jnp.concatenate([edge_index[1].astype(jnp.int32), jnp.asarray(pad_dst)])
    src = src.reshape(NC, NS, K, EB)
    dst = dst.reshape(NC, NS, K, EB)
    zeros = jnp.zeros((ROWS_PER_TILE, D), jnp.float32)
    h = jnp.concatenate([x, jnp.zeros((NP - N, D), x.dtype)])
    batch_p = jnp.concatenate(
        [batch.astype(jnp.int32),
         jnp.full((NP - N,), G, jnp.int32)]).reshape(NP // R, 1, R)

    parts = _agg(h, src, dst, zeros)
    h = _mlp(h, parts, W1_0, b1_0.reshape(1, D), W2_0, b2_0.reshape(1, D))
    parts = _agg(h, src, dst, zeros)
    h = _mlp(h, parts, W1_1, b1_1.reshape(1, D), W2_1, b2_1.reshape(1, D))
    parts = _agg(h, src, dst, zeros)
    out = _mlp_pool(h, parts, batch_p,
                    W1_2, b1_2.reshape(1, D), W2_2, b2_2.reshape(1, D),
                    Wf1, bf1.reshape(1, D), Wf2, bf2.reshape(1, D))
    return out


# speedup vs baseline: 11.3001x; 1.4853x over previous
import functools

import jax
import jax.numpy as jnp
import numpy as np
from jax import lax
from jax.experimental import pallas as pl
from jax.experimental.pallas import tpu as pltpu
from jax.experimental.pallas import tpu_sc as plsc

N = 10000
E = 320000
D = 128
G = 64
NC, NS = 2, 16
NP = 10240
R = 1024
EB = 128
K = 80
KH = K // 2
EP = NC * NS * K * EB
ROWS_PER_TILE = NP // NS


def _agg_body(h_hbm, src_hbm, dst_hbm, zeros_hbm, out_hbm,
              acc, src_v, dst_v, r0, r1, g0, g1, s0, s1):
    cid = lax.axis_index("c")
    sid = lax.axis_index("s")
    row0 = sid * ROWS_PER_TILE
    pltpu.sync_copy(zeros_hbm.at[pl.ds(row0, ROWS_PER_TILE)],
                    acc.at[pl.ds(row0, ROWS_PER_TILE)])
    plsc.subcore_barrier()

    def gather(j, rv, sem):
        pltpu.make_async_copy(h_hbm.at[src_v.at[j]], rv, sem).start()

    def gather_wait(rv, sem):
        pltpu.make_async_copy(h_hbm.at[src_v.at[0]], rv, sem).wait()

    def scatter(j, rv, sem):
        pltpu.make_async_copy(rv, acc.at[dst_v.at[j]], sem).start(add=True)

    def scatter_wait(rv, sem):
        pltpu.make_async_copy(rv, acc.at[dst_v.at[0]], sem).wait()

    for half in range(2):
        pltpu.sync_copy(src_hbm.at[cid, sid, pl.ds(half * KH, KH)], src_v)
        pltpu.sync_copy(dst_hbm.at[cid, sid, pl.ds(half * KH, KH)], dst_v)
        gather(0, r0, g0)

        def step(t, carry):
            j0 = 2 * t
            j1 = j0 + 1

            @pl.when(t > 0)
            def _():
                scatter_wait(r1, s1)

            gather(j1, r1, g1)
            gather_wait(r0, g0)
            scatter(j0, r0, s0)

            @pl.when(t < KH // 2 - 1)
            def _():
                scatter_wait(r0, s0)
                gather(j0 + 2, r0, g0)

            gather_wait(r1, g1)
            scatter(j1, r1, s1)
            return carry

        lax.fori_loop(0, KH // 2, step, 0)
        scatter_wait(r0, s0)
        scatter_wait(r1, s1)
    plsc.subcore_barrier()
    pltpu.sync_copy(acc.at[pl.ds(row0, ROWS_PER_TILE)],
                    out_hbm.at[cid, pl.ds(row0, ROWS_PER_TILE)])


_agg = functools.partial(
    pl.kernel,
    out_type=jax.ShapeDtypeStruct((NC, NP, D), jnp.float32),
    mesh=plsc.VectorSubcoreMesh(core_axis_name="c", subcore_axis_name="s"),
    scratch_types=[
        pltpu.VMEM_SHARED((NP, D), jnp.float32),
        pltpu.VMEM((KH, EB), jnp.int32),
        pltpu.VMEM((KH, EB), jnp.int32),
        pltpu.VMEM((EB, D), jnp.float32),
        pltpu.VMEM((EB, D), jnp.float32),
        pltpu.SemaphoreType.DMA,
        pltpu.SemaphoreType.DMA,
        pltpu.SemaphoreType.DMA,
        pltpu.SemaphoreType.DMA,
    ],
)(_agg_body)


def _mlp_body(h_ref, p_ref, w1_ref, b1_ref, w2_ref, b2_ref, o_ref):
    z = h_ref[...] + p_ref[0] + p_ref[1]
    z = jnp.maximum(
        jnp.dot(z, w1_ref[...], preferred_element_type=jnp.float32) + b1_ref[...],
        0.0)
    z = jnp.dot(z, w2_ref[...], preferred_element_type=jnp.float32) + b2_ref[...]
    o_ref[...] = jnp.maximum(z, 0.0)


_mlp = pl.pallas_call(
    _mlp_body,
    grid=(NP // R,),
    in_specs=[
        pl.BlockSpec((R, D), lambda i: (i, 0)),
        pl.BlockSpec((NC, R, D), lambda i: (0, i, 0)),
        pl.BlockSpec((D, D), lambda i: (0, 0)),
        pl.BlockSpec((1, D), lambda i: (0, 0)),
        pl.BlockSpec((D, D), lambda i: (0, 0)),
        pl.BlockSpec((1, D), lambda i: (0, 0)),
    ],
    out_specs=pl.BlockSpec((R, D), lambda i: (i, 0)),
    out_shape=jax.ShapeDtypeStruct((NP, D), jnp.float32),
)


def _mlp_pool_body(h_ref, p_ref, batch_ref, w1_ref, b1_ref, w2_ref, b2_ref,
                   wf1_ref, bf1_ref, wf2_ref, bf2_ref, o_ref, pooled):
    i = pl.program_id(0)
    z = h_ref[...] + p_ref[0] + p_ref[1]
    z = jnp.maximum(
        jnp.dot(z, w1_ref[...], preferred_element_type=jnp.float32) + b1_ref[...],
        0.0)
    z = jnp.dot(z, w2_ref[...], preferred_element_type=jnp.float32) + b2_ref[...]
    h3 = jnp.maximum(z, 0.0)
    seg = batch_ref[0, 0, :]
    m = (lax.broadcasted_iota(jnp.int32, (G, R), 0) == seg[None, :]
         ).astype(jnp.float32)
    part = jnp.dot(m, h3, preferred_element_type=jnp.float32)

    @pl.when(i == 0)
    def _():
        pooled[...] = part

    @pl.when(i > 0)
    def _():
        pooled[...] += part

    @pl.when(i == pl.num_programs(0) - 1)
    def _():
        q = jnp.maximum(
            jnp.dot(pooled[...], wf1_ref[...],
                    preferred_element_type=jnp.float32) + bf1_ref[...],
            0.0)
        o_ref[...] = (jnp.dot(q, wf2_ref[...],
                              preferred_element_type=jnp.float32)
                      + bf2_ref[...])


_mlp_pool = pl.pallas_call(
    _mlp_pool_body,
    grid=(NP // R,),
    in_specs=[
        pl.BlockSpec((R, D), lambda i: (i, 0)),
        pl.BlockSpec((NC, R, D), lambda i: (0, i, 0)),
        pl.BlockSpec((1, 1, R), lambda i: (i, 0, 0)),
        pl.BlockSpec((D, D), lambda i: (0, 0)),
        pl.BlockSpec((1, D), lambda i: (0, 0)),
        pl.BlockSpec((D, D), lambda i: (0, 0)),
        pl.BlockSpec((1, D), lambda i: (0, 0)),
        pl.BlockSpec((D, D), lambda i: (0, 0)),
        pl.BlockSpec((1, D), lambda i: (0, 0)),
        pl.BlockSpec((D, D), lambda i: (0, 0)),
        pl.BlockSpec((1, D), lambda i: (0, 0)),
    ],
    out_specs=pl.BlockSpec((G, D), lambda i: (0, 0)),
    out_shape=jax.ShapeDtypeStruct((G, D), jnp.float32),
    scratch_shapes=[pltpu.VMEM((G, D), jnp.float32)],
)


def kernel(x, edge_index, edge_attr, batch,
           W1_0, b1_0, W2_0, b2_0, W1_1, b1_1, W2_1, b2_1,
           W1_2, b1_2, W2_2, b2_2, Wf1, bf1, Wf2, bf2):
    del edge_attr
    pad = EP - E
    pad_src = (np.arange(pad, dtype=np.int32) * 97) % N
    pad_dst = N + (np.arange(pad, dtype=np.int32) % (NP - N))
    src = jnp.concatenate([edge_index[0].astype(jnp.int32), jnp.asarray(pad_src)])
    dst = jnp.concatenate([edge_index[1].astype(jnp.int32), jnp.asarray(pad_dst)])
    src = src.reshape(NC, NS, K, EB)
    dst = dst.reshape(NC, NS, K, EB)
    zeros = jnp.zeros((NP, D), jnp.float32)
    h = jnp.concatenate([x, jnp.zeros((NP - N, D), x.dtype)])
    batch_p = jnp.concatenate(
        [batch.astype(jnp.int32),
         jnp.full((NP - N,), G, jnp.int32)]).reshape(NP // R, 1, R)

    parts = _agg(h, src, dst, zeros)
    h = _mlp(h, parts, W1_0, b1_0.reshape(1, D), W2_0, b2_0.reshape(1, D))
    parts = _agg(h, src, dst, zeros)
    h = _mlp(h, parts, W1_1, b1_1.reshape(1, D), W2_1, b2_1.reshape(1, D))
    parts = _agg(h, src, dst, zeros)
    out = _mlp_pool(h, parts, batch_p,
                    W1_2, b1_2.reshape(1, D), W2_2, b2_2.reshape(1, D),
                    Wf1, bf1.reshape(1, D), Wf2, bf2.reshape(1, D))
    return out
